# baseline (device time: 33015 ns/iter reference)
import jax
import jax.numpy as jnp
from jax import lax
from jax.experimental import pallas as pl
from jax.experimental.pallas import tpu as pltpu

N_DEV = 4
N_CHUNK = 4

_SEND_ORDER = (2, 1, 3)
_WAIT_ORDER = (1, 3, 2)


def kernel(A, B):
    m, k = A.shape
    k2, n = B.shape
    assert k == k2
    mq = m // N_DEV
    nc = n // N_CHUNK

    def body(a_hbm, b_hbm, out_ref, a_vmem, b_vmem, a_bf, pbuf, rs_recv,
             ag_buf, in_sems, rs_send_sems, rs_recv_sems, ag_send_sems,
             ag_recv_sems, own_sems):
        my = lax.axis_index("i")

        a_cp = pltpu.make_async_copy(a_hbm, a_vmem, in_sems.at[0])
        b_cp = pltpu.make_async_copy(b_hbm, b_vmem, in_sems.at[1])
        a_cp.start()
        b_cp.start()

        barrier_sem = pltpu.get_barrier_semaphore()
        for d in range(1, N_DEV):
            pl.semaphore_signal(
                barrier_sem, inc=1,
                device_id=((my + d) % N_DEV,),
                device_id_type=pl.DeviceIdType.MESH,
            )
        pl.semaphore_wait(barrier_sem, N_DEV - 1)

        a_cp.wait()
        b_cp.wait()
        a_bf[:, :] = a_vmem[:, :].astype(jnp.bfloat16)

        rs_rdmas = {}
        own = [None] * N_CHUNK
        for c in range(N_CHUNK):
            bc = b_vmem[:, pl.ds(c * nc, nc)].astype(jnp.bfloat16)
            for d in _SEND_ORDER:
                q = (my + d) % N_DEV
                pbuf[c, d - 1, :, :] = jnp.dot(
                    a_bf[pl.ds(q * mq, mq), :], bc,
                    preferred_element_type=jnp.float32,
                ).astype(jnp.bfloat16)
                rdma = pltpu.make_async_remote_copy(
                    src_ref=pbuf.at[c, d - 1],
                    dst_ref=rs_recv.at[c, d - 1],
                    send_sem=rs_send_sems.at[c, d - 1],
                    recv_sem=rs_recv_sems.at[c, d - 1],
                    device_id=(q,),
                    device_id_type=pl.DeviceIdType.MESH,
                )
                rdma.start()
                rs_rdmas[(c, d)] = rdma
            own[c] = jnp.dot(
                a_bf[pl.ds(my * mq, mq), :], bc,
                preferred_element_type=jnp.float32,
            )

        ag_send_rdmas = {}
        own_copies = []
        for c in range(N_CHUNK):
            acc = own[c]
            for d in _WAIT_ORDER:
                rs_rdmas[(c, d)].wait_recv()
                acc = acc + rs_recv[c, d - 1, :, :].astype(jnp.float32)
            ag_buf[c, :, :] = jnp.maximum(acc, 0.0).astype(jnp.bfloat16)
            cp = pltpu.make_async_copy(
                ag_buf.at[c],
                out_ref.at[pl.ds(my * mq, mq), pl.ds(c * nc, nc)],
                own_sems.at[c],
            )
            cp.start()
            own_copies.append(cp)
            for d in _SEND_ORDER:
                rdma = pltpu.make_async_remote_copy(
                    src_ref=ag_buf.at[c],
                    dst_ref=out_ref.at[pl.ds(my * mq, mq), pl.ds(c * nc, nc)],
                    send_sem=ag_send_sems.at[c, d - 1],
                    recv_sem=ag_recv_sems.at[c, d - 1],
                    device_id=((my + d) % N_DEV,),
                    device_id_type=pl.DeviceIdType.MESH,
                )
                rdma.start()
                ag_send_rdmas[(c, d)] = rdma

        for c in range(N_CHUNK):
            for d in _WAIT_ORDER:
                s = (my - d) % N_DEV
                recv = pltpu.make_async_remote_copy(
                    src_ref=ag_buf.at[c],
                    dst_ref=out_ref.at[pl.ds(s * mq, mq), pl.ds(c * nc, nc)],
                    send_sem=ag_send_sems.at[c, d - 1],
                    recv_sem=ag_recv_sems.at[c, d - 1],
                    device_id=(s,),
                    device_id_type=pl.DeviceIdType.MESH,
                )
                recv.wait_recv()

        for cp in own_copies:
            cp.wait()
        for c in range(N_CHUNK):
            for d in range(1, N_DEV):
                rs_rdmas[(c, d)].wait_send()
                ag_send_rdmas[(c, d)].wait_send()

    return pl.pallas_call(
        body,
        out_shape=jax.ShapeDtypeStruct((m, n), jnp.bfloat16),
        in_specs=[
            pl.BlockSpec(memory_space=pl.ANY),
            pl.BlockSpec(memory_space=pl.ANY),
        ],
        out_specs=pl.BlockSpec(memory_space=pl.ANY),
        scratch_shapes=[
            pltpu.VMEM((m, k), jnp.float32),
            pltpu.VMEM((k, n), jnp.float32),
            pltpu.VMEM((m, k), jnp.bfloat16),
            pltpu.VMEM((N_CHUNK, N_DEV - 1, mq, nc), jnp.bfloat16),
            pltpu.VMEM((N_CHUNK, N_DEV - 1, mq, nc), jnp.bfloat16),
            pltpu.VMEM((N_CHUNK, mq, nc), jnp.bfloat16),
            pltpu.SemaphoreType.DMA((2,)),
            pltpu.SemaphoreType.DMA((N_CHUNK, N_DEV - 1)),
            pltpu.SemaphoreType.DMA((N_CHUNK, N_DEV - 1)),
            pltpu.SemaphoreType.DMA((N_CHUNK, N_DEV - 1)),
            pltpu.SemaphoreType.DMA((N_CHUNK, N_DEV - 1)),
            pltpu.SemaphoreType.DMA((N_CHUNK,)),
        ],
        compiler_params=pltpu.CompilerParams(collective_id=0),
    )(A, B)


# device time: 31631 ns/iter; 1.0438x vs baseline; 1.0438x over previous
import jax
import jax.numpy as jnp
from jax import lax
from jax.experimental import pallas as pl
from jax.experimental.pallas import tpu as pltpu

N_DEV = 4
N_CHUNK = 2


def kernel(A, B):
    m, k = A.shape
    k2, n = B.shape
    assert k == k2
    mq = m // N_DEV
    nc = n // N_CHUNK

    def body(a_hbm, b_hbm, out_ref, a_vmem, b_vmem, a_bf,
             dflow_send, dflow_recv, direct_send, direct_recv,
             comb_send, comb_recv, ag_buf,
             in_sems, dflow_s, dflow_r, direct_s, direct_r,
             comb_s, comb_r, agr_s, agr_r, agl_s, agl_r,
             fwd_s, fwd_r, own_sems):
        my = lax.axis_index("i")
        right = (my + 1) % N_DEV
        left = (my - 1) % N_DEV

        a_cp = pltpu.make_async_copy(a_hbm, a_vmem, in_sems.at[0])
        b_cp = pltpu.make_async_copy(b_hbm, b_vmem, in_sems.at[1])
        a_cp.start()
        b_cp.start()

        barrier_sem = pltpu.get_barrier_semaphore()
        for nbr in (left, right):
            pl.semaphore_signal(
                barrier_sem, inc=1,
                device_id=(nbr,), device_id_type=pl.DeviceIdType.MESH,
            )
        pl.semaphore_wait(barrier_sem, 2)

        a_cp.wait()
        b_cp.wait()
        a_bf[:, :] = a_vmem[:, :].astype(jnp.bfloat16)

        def quarter_rows(q):
            return pl.ds(q * mq, mq)

        def out_slice(q, h):
            return out_ref.at[pl.ds(q * mq, mq), pl.ds(h * nc, nc)]

        dflow_rd, direct_rd, comb_rd = {}, {}, {}
        agr_rd, agl_rd, fwd_rd = {}, {}, {}
        own_c = [None] * N_CHUNK
        own_cm1 = [None] * N_CHUNK

        for h in range(N_CHUNK):
            bh = b_vmem[:, pl.ds(h * nc, nc)].astype(jnp.bfloat16)
            dflow_send[h, :, :] = jnp.dot(
                a_bf[quarter_rows((my + 2) % N_DEV), :], bh,
                preferred_element_type=jnp.float32,
            ).astype(jnp.bfloat16)
            rdma = pltpu.make_async_remote_copy(
                src_ref=dflow_send.at[h], dst_ref=dflow_recv.at[h],
                send_sem=dflow_s.at[h], recv_sem=dflow_r.at[h],
                device_id=(left,), device_id_type=pl.DeviceIdType.MESH,
            )
            rdma.start()
            dflow_rd[h] = rdma

            direct_send[h, :, :] = jnp.dot(
                a_bf[quarter_rows(right), :], bh,
                preferred_element_type=jnp.float32,
            ).astype(jnp.bfloat16)
            rdma = pltpu.make_async_remote_copy(
                src_ref=direct_send.at[h], dst_ref=direct_recv.at[h],
                send_sem=direct_s.at[h], recv_sem=direct_r.at[h],
                device_id=(right,), device_id_type=pl.DeviceIdType.MESH,
            )
            rdma.start()
            direct_rd[h] = rdma

            own_cm1[h] = jnp.dot(
                a_bf[quarter_rows(left), :], bh,
                preferred_element_type=jnp.float32,
            )
            own_c[h] = jnp.dot(
                a_bf[quarter_rows(my), :], bh,
                preferred_element_type=jnp.float32,
            )

        for h in range(N_CHUNK):
            dflow_rd[h].wait_recv()
            comb_send[h, :, :] = (
                own_cm1[h] + dflow_recv[h, :, :].astype(jnp.float32)
            ).astype(jnp.bfloat16)
            rdma = pltpu.make_async_remote_copy(
                src_ref=comb_send.at[h], dst_ref=comb_recv.at[h],
                send_sem=comb_s.at[h], recv_sem=comb_r.at[h],
                device_id=(left,), device_id_type=pl.DeviceIdType.MESH,
            )
            rdma.start()
            comb_rd[h] = rdma

        own_copies = []
        for h in range(N_CHUNK):
            comb_rd[h].wait_recv()
            direct_rd[h].wait_recv()
            r = jnp.maximum(
                own_c[h]
                + comb_recv[h, :, :].astype(jnp.float32)
                + direct_recv[h, :, :].astype(jnp.float32),
                0.0,
            )
            ag_buf[h, :, :] = r.astype(jnp.bfloat16)
            cp = pltpu.make_async_copy(
                ag_buf.at[h], out_slice(my, h), own_sems.at[h]
            )
            cp.start()
            own_copies.append(cp)
            rdma = pltpu.make_async_remote_copy(
                src_ref=ag_buf.at[h], dst_ref=out_slice(my, h),
                send_sem=agr_s.at[h], recv_sem=agr_r.at[h],
                device_id=(right,), device_id_type=pl.DeviceIdType.MESH,
            )
            rdma.start()
            agr_rd[h] = rdma
            rdma = pltpu.make_async_remote_copy(
                src_ref=ag_buf.at[h], dst_ref=out_slice(my, h),
                send_sem=agl_s.at[h], recv_sem=agl_r.at[h],
                device_id=(left,), device_id_type=pl.DeviceIdType.MESH,
            )
            rdma.start()
            agl_rd[h] = rdma

        for h in range(N_CHUNK):
            recv = pltpu.make_async_remote_copy(
                src_ref=ag_buf.at[h],
                dst_ref=out_slice(left, h),
                send_sem=agr_s.at[h], recv_sem=agr_r.at[h],
                device_id=(left,), device_id_type=pl.DeviceIdType.MESH,
            )
            recv.wait_recv()
            rdma = pltpu.make_async_remote_copy(
                src_ref=out_slice(left, h), dst_ref=out_slice(left, h),
                send_sem=fwd_s.at[h], recv_sem=fwd_r.at[h],
                device_id=(right,), device_id_type=pl.DeviceIdType.MESH,
            )
            rdma.start()
            fwd_rd[h] = rdma

        for h in range(N_CHUNK):
            recv = pltpu.make_async_remote_copy(
                src_ref=ag_buf.at[h],
                dst_ref=out_slice(right, h),
                send_sem=agl_s.at[h], recv_sem=agl_r.at[h],
                device_id=(right,), device_id_type=pl.DeviceIdType.MESH,
            )
            recv.wait_recv()
            recv = pltpu.make_async_remote_copy(
                src_ref=ag_buf.at[h],
                dst_ref=out_slice((my + 2) % N_DEV, h),
                send_sem=fwd_s.at[h], recv_sem=fwd_r.at[h],
                device_id=(left,), device_id_type=pl.DeviceIdType.MESH,
            )
            recv.wait_recv()

        for cp in own_copies:
            cp.wait()
        for h in range(N_CHUNK):
            dflow_rd[h].wait_send()
            direct_rd[h].wait_send()
            comb_rd[h].wait_send()
            agr_rd[h].wait_send()
            agl_rd[h].wait_send()
            fwd_rd[h].wait_send()

    chunk_buf = pltpu.VMEM((N_CHUNK, mq, nc), jnp.bfloat16)
    return pl.pallas_call(
        body,
        out_shape=jax.ShapeDtypeStruct((m, n), jnp.bfloat16),
        in_specs=[
            pl.BlockSpec(memory_space=pl.ANY),
            pl.BlockSpec(memory_space=pl.ANY),
        ],
        out_specs=pl.BlockSpec(memory_space=pl.ANY),
        scratch_shapes=[
            pltpu.VMEM((m, k), jnp.float32),
            pltpu.VMEM((k, n), jnp.float32),
            pltpu.VMEM((m, k), jnp.bfloat16),
            chunk_buf,
            chunk_buf,
            chunk_buf,
            chunk_buf,
            chunk_buf,
            chunk_buf,
            chunk_buf,
            pltpu.SemaphoreType.DMA((2,)),
            pltpu.SemaphoreType.DMA((N_CHUNK,)),
            pltpu.SemaphoreType.DMA((N_CHUNK,)),
            pltpu.SemaphoreType.DMA((N_CHUNK,)),
            pltpu.SemaphoreType.DMA((N_CHUNK,)),
            pltpu.SemaphoreType.DMA((N_CHUNK,)),
            pltpu.SemaphoreType.DMA((N_CHUNK,)),
            pltpu.SemaphoreType.DMA((N_CHUNK,)),
            pltpu.SemaphoreType.DMA((N_CHUNK,)),
            pltpu.SemaphoreType.DMA((N_CHUNK,)),
            pltpu.SemaphoreType.DMA((N_CHUNK,)),
            pltpu.SemaphoreType.DMA((N_CHUNK,)),
            pltpu.SemaphoreType.DMA((N_CHUNK,)),
            pltpu.SemaphoreType.DMA((N_CHUNK,)),
        ],
        compiler_params=pltpu.CompilerParams(collective_id=0),
    )(A, B)


# device time: 28729 ns/iter; 1.1492x vs baseline; 1.1010x over previous
import jax
import jax.numpy as jnp
from jax import lax
from jax.experimental import pallas as pl
from jax.experimental.pallas import tpu as pltpu

N_DEV = 4
N_CHUNK = 4


def kernel(A, B):
    m, k = A.shape
    k2, n = B.shape
    assert k == k2
    mq = m // N_DEV
    nc = n // N_CHUNK

    def body(a_hbm, b_hbm, out_ref, a_vmem, b_vmem, a_bf,
             dflow_send, dflow_recv, direct_send, direct_recv,
             comb_send, comb_recv, ag_buf,
             in_sems, dflow_s, dflow_r, direct_s, direct_r,
             comb_s, comb_r, agr_s, agr_r, agl_s, agl_r,
             fwd_s, fwd_r, own_sems):
        my = lax.axis_index("i")
        right = (my + 1) % N_DEV
        left = (my - 1) % N_DEV

        a_cp = pltpu.make_async_copy(a_hbm, a_vmem, in_sems.at[0])
        a_cp.start()
        b_cps = []
        for h in range(N_CHUNK):
            b_cp = pltpu.make_async_copy(
                b_hbm.at[:, pl.ds(h * nc, nc)],
                b_vmem.at[:, pl.ds(h * nc, nc)],
                in_sems.at[1 + h],
            )
            b_cp.start()
            b_cps.append(b_cp)

        barrier_sem = pltpu.get_barrier_semaphore()
        for nbr in (left, right):
            pl.semaphore_signal(
                barrier_sem, inc=1,
                device_id=(nbr,), device_id_type=pl.DeviceIdType.MESH,
            )
        pl.semaphore_wait(barrier_sem, 2)

        a_cp.wait()
        a_bf[:, :] = a_vmem[:, :].astype(jnp.bfloat16)

        def quarter_rows(q):
            return pl.ds(q * mq, mq)

        def out_slice(q, h):
            return out_ref.at[pl.ds(q * mq, mq), pl.ds(h * nc, nc)]

        dflow_rd, direct_rd, comb_rd = {}, {}, {}
        agr_rd, agl_rd, fwd_rd = {}, {}, {}
        own_c = [None] * N_CHUNK
        own_cm1 = [None] * N_CHUNK

        for h in range(N_CHUNK):
            b_cps[h].wait()
            bh = b_vmem[:, pl.ds(h * nc, nc)].astype(jnp.bfloat16)
            dflow_send[h, :, :] = jnp.dot(
                a_bf[quarter_rows((my + 2) % N_DEV), :], bh,
                preferred_element_type=jnp.float32,
            ).astype(jnp.bfloat16)
            rdma = pltpu.make_async_remote_copy(
                src_ref=dflow_send.at[h], dst_ref=dflow_recv.at[h],
                send_sem=dflow_s.at[h], recv_sem=dflow_r.at[h],
                device_id=(left,), device_id_type=pl.DeviceIdType.MESH,
            )
            rdma.start()
            dflow_rd[h] = rdma

            direct_send[h, :, :] = jnp.dot(
                a_bf[quarter_rows(right), :], bh,
                preferred_element_type=jnp.float32,
            ).astype(jnp.bfloat16)
            rdma = pltpu.make_async_remote_copy(
                src_ref=direct_send.at[h], dst_ref=direct_recv.at[h],
                send_sem=direct_s.at[h], recv_sem=direct_r.at[h],
                device_id=(right,), device_id_type=pl.DeviceIdType.MESH,
            )
            rdma.start()
            direct_rd[h] = rdma

            own_cm1[h] = jnp.dot(
                a_bf[quarter_rows(left), :], bh,
                preferred_element_type=jnp.float32,
            )
            own_c[h] = jnp.dot(
                a_bf[quarter_rows(my), :], bh,
                preferred_element_type=jnp.float32,
            )

        for h in range(N_CHUNK):
            dflow_rd[h].wait_recv()
            comb_send[h, :, :] = (
                own_cm1[h] + dflow_recv[h, :, :].astype(jnp.float32)
            ).astype(jnp.bfloat16)
            rdma = pltpu.make_async_remote_copy(
                src_ref=comb_send.at[h], dst_ref=comb_recv.at[h],
                send_sem=comb_s.at[h], recv_sem=comb_r.at[h],
                device_id=(left,), device_id_type=pl.DeviceIdType.MESH,
            )
            rdma.start()
            comb_rd[h] = rdma

        own_copies = []
        for h in range(N_CHUNK):
            comb_rd[h].wait_recv()
            direct_rd[h].wait_recv()
            r = jnp.maximum(
                own_c[h]
                + comb_recv[h, :, :].astype(jnp.float32)
                + direct_recv[h, :, :].astype(jnp.float32),
                0.0,
            )
            ag_buf[h, :, :] = r.astype(jnp.bfloat16)
            cp = pltpu.make_async_copy(
                ag_buf.at[h], out_slice(my, h), own_sems.at[h]
            )
            cp.start()
            own_copies.append(cp)
            rdma = pltpu.make_async_remote_copy(
                src_ref=ag_buf.at[h], dst_ref=out_slice(my, h),
                send_sem=agr_s.at[h], recv_sem=agr_r.at[h],
                device_id=(right,), device_id_type=pl.DeviceIdType.MESH,
            )
            rdma.start()
            agr_rd[h] = rdma
            rdma = pltpu.make_async_remote_copy(
                src_ref=ag_buf.at[h], dst_ref=out_slice(my, h),
                send_sem=agl_s.at[h], recv_sem=agl_r.at[h],
                device_id=(left,), device_id_type=pl.DeviceIdType.MESH,
            )
            rdma.start()
            agl_rd[h] = rdma

        for h in range(N_CHUNK):
            recv = pltpu.make_async_remote_copy(
                src_ref=ag_buf.at[h],
                dst_ref=out_slice(left, h),
                send_sem=agr_s.at[h], recv_sem=agr_r.at[h],
                device_id=(left,), device_id_type=pl.DeviceIdType.MESH,
            )
            recv.wait_recv()
            rdma = pltpu.make_async_remote_copy(
                src_ref=out_slice(left, h), dst_ref=out_slice(left, h),
                send_sem=fwd_s.at[h], recv_sem=fwd_r.at[h],
                device_id=(right,), device_id_type=pl.DeviceIdType.MESH,
            )
            rdma.start()
            fwd_rd[h] = rdma

        for h in range(N_CHUNK):
            recv = pltpu.make_async_remote_copy(
                src_ref=ag_buf.at[h],
                dst_ref=out_slice(right, h),
                send_sem=agl_s.at[h], recv_sem=agl_r.at[h],
                device_id=(right,), device_id_type=pl.DeviceIdType.MESH,
            )
            recv.wait_recv()
            recv = pltpu.make_async_remote_copy(
                src_ref=ag_buf.at[h],
                dst_ref=out_slice((my + 2) % N_DEV, h),
                send_sem=fwd_s.at[h], recv_sem=fwd_r.at[h],
                device_id=(left,), device_id_type=pl.DeviceIdType.MESH,
            )
            recv.wait_recv()

        for cp in own_copies:
            cp.wait()
        for h in range(N_CHUNK):
            dflow_rd[h].wait_send()
            direct_rd[h].wait_send()
            comb_rd[h].wait_send()
            agr_rd[h].wait_send()
            agl_rd[h].wait_send()
            fwd_rd[h].wait_send()

    chunk_buf = pltpu.VMEM((N_CHUNK, mq, nc), jnp.bfloat16)
    return pl.pallas_call(
        body,
        out_shape=jax.ShapeDtypeStruct((m, n), jnp.bfloat16),
        in_specs=[
            pl.BlockSpec(memory_space=pl.ANY),
            pl.BlockSpec(memory_space=pl.ANY),
        ],
        out_specs=pl.BlockSpec(memory_space=pl.ANY),
        scratch_shapes=[
            pltpu.VMEM((m, k), jnp.float32),
            pltpu.VMEM((k, n), jnp.float32),
            pltpu.VMEM((m, k), jnp.bfloat16),
            chunk_buf,
            chunk_buf,
            chunk_buf,
            chunk_buf,
            chunk_buf,
            chunk_buf,
            chunk_buf,
            pltpu.SemaphoreType.DMA((1 + N_CHUNK,)),
            pltpu.SemaphoreType.DMA((N_CHUNK,)),
            pltpu.SemaphoreType.DMA((N_CHUNK,)),
            pltpu.SemaphoreType.DMA((N_CHUNK,)),
            pltpu.SemaphoreType.DMA((N_CHUNK,)),
            pltpu.SemaphoreType.DMA((N_CHUNK,)),
            pltpu.SemaphoreType.DMA((N_CHUNK,)),
            pltpu.SemaphoreType.DMA((N_CHUNK,)),
            pltpu.SemaphoreType.DMA((N_CHUNK,)),
            pltpu.SemaphoreType.DMA((N_CHUNK,)),
            pltpu.SemaphoreType.DMA((N_CHUNK,)),
            pltpu.SemaphoreType.DMA((N_CHUNK,)),
            pltpu.SemaphoreType.DMA((N_CHUNK,)),
            pltpu.SemaphoreType.DMA((N_CHUNK,)),
        ],
        compiler_params=pltpu.CompilerParams(collective_id=0),
    )(A, B)
